# Initial kernel scaffold; baseline (speedup 1.0000x reference)
import jax
import jax.numpy as jnp
from jax.experimental import pallas as pl


def kernel(seq1, adj, W1, b1, a1, W2, b2, a2, sparse):
    raise NotImplementedError("write your pallas kernel here")


# speedup vs baseline: 1.0099x; 1.0099x over previous
import jax
import jax.numpy as jnp
from jax.experimental import pallas as pl


def _pick_bm(n: int) -> int:
    for bm in (512, 400, 256, 200, 128, 80, 40, 16, 8):
        if n % bm == 0:
            return bm
    return n


def _layer1_kernel(adj_ref, seq_ref, w1_ref, b1_ref, a1_ref, w2_ref, x2_ref):
    t = jnp.dot(adj_ref[...], seq_ref[...], preferred_element_type=jnp.float32)
    h = jnp.dot(t, w1_ref[...], preferred_element_type=jnp.float32) + b1_ref[...]
    h = jnp.where(h >= 0, h, a1_ref[...] * h)
    x2_ref[...] = jnp.dot(h, w2_ref[...], preferred_element_type=jnp.float32)


def _layer2_kernel(adj_ref, x2_ref, b2_ref, a2_ref, out_ref):
    t = jnp.dot(adj_ref[...], x2_ref[...], preferred_element_type=jnp.float32)
    t = t + b2_ref[...]
    out_ref[...] = jnp.where(t >= 0, t, a2_ref[...] * t)


def kernel(seq1, adj, W1, b1, a1, W2, b2, a2, sparse):
    n = adj.shape[-1]
    d_in = seq1.shape[-1]
    d_h = W1.shape[-1]
    d_out = W2.shape[-1]
    bm = _pick_bm(n)
    grid = (n // bm,)

    adj2 = adj[0]
    seq = seq1[0]
    b1r = jnp.broadcast_to(b1.reshape(1, d_h), (1, d_h))
    a1r = jnp.broadcast_to(a1.reshape(1, 1), (1, d_h))
    b2r = jnp.broadcast_to(b2.reshape(1, d_out), (1, d_out))
    a2r = jnp.broadcast_to(a2.reshape(1, 1), (1, d_out))

    row_spec = pl.BlockSpec((bm, n), lambda i: (i, 0))
    full = lambda shape: pl.BlockSpec(shape, lambda i: (0,) * len(shape))

    x2 = pl.pallas_call(
        _layer1_kernel,
        grid=grid,
        in_specs=[
            row_spec,
            full((n, d_in)),
            full((d_in, d_h)),
            full((1, d_h)),
            full((1, d_h)),
            full((d_h, d_out)),
        ],
        out_specs=pl.BlockSpec((bm, d_out), lambda i: (i, 0)),
        out_shape=jax.ShapeDtypeStruct((n, d_out), jnp.float32),
    )(adj2, seq, W1, b1r, a1r, W2)

    out = pl.pallas_call(
        _layer2_kernel,
        grid=grid,
        in_specs=[
            row_spec,
            full((n, d_out)),
            full((1, d_out)),
            full((1, d_out)),
        ],
        out_specs=pl.BlockSpec((bm, d_out), lambda i: (i, 0)),
        out_shape=jax.ShapeDtypeStruct((n, d_out), jnp.float32),
    )(adj2, x2, b2r, a2r)

    return out[None]
